# Initial kernel scaffold; baseline (speedup 1.0000x reference)
import jax
import jax.numpy as jnp
from jax.experimental import pallas as pl


def kernel(states, partitions):
    raise NotImplementedError("write your pallas kernel here")


# speedup vs baseline: 2.7034x; 2.7034x over previous
import jax
import jax.numpy as jnp
from jax import lax
from jax.experimental import pallas as pl
from jax.experimental.pallas import tpu as pltpu
from jax.experimental.pallas import tpu_sc as plsc

T = 65536
D = 512
NB = 10
NC, NS, L = 2, 16, 16
NW = NC * NS
RPW = T // NW
NSER = 8
NPART = 4
NBINS2 = NB * NB


def _stage1_body(st_ref, m_ref, sums_ref, min_ref, max_ref):
    i = pl.program_id(0)
    m = m_ref[...]
    na = jnp.sum(m, axis=1, keepdims=True)
    s = lax.dot_general(m, st_ref[...], (((1,), (1,)), ((), ())),
                        preferred_element_type=jnp.float32,
                        precision=lax.Precision.HIGHEST)
    s = s / na
    sums_ref[0] = s
    bmin = jnp.min(s, axis=1)[None, :]
    bmax = jnp.max(s, axis=1)[None, :]

    @pl.when(i == 0)
    def _():
        min_ref[...] = bmin
        max_ref[...] = bmax

    @pl.when(i != 0)
    def _():
        min_ref[...] = jnp.minimum(min_ref[...], bmin)
        max_ref[...] = jnp.maximum(max_ref[...], bmax)


_stage1 = pl.pallas_call(
    _stage1_body,
    grid=(NW,),
    in_specs=[pl.BlockSpec((RPW, D), lambda i: (i, 0)),
              pl.BlockSpec((NSER, D), lambda i: (0, 0))],
    out_specs=[pl.BlockSpec((1, NSER, RPW), lambda i: (i, 0, 0)),
               pl.BlockSpec((1, NSER), lambda i: (0, 0)),
               pl.BlockSpec((1, NSER), lambda i: (0, 0))],
    out_shape=[jax.ShapeDtypeStruct((NW, NSER, RPW), jnp.float32),
               jax.ShapeDtypeStruct((1, NSER), jnp.float32),
               jax.ShapeDtypeStruct((1, NSER), jnp.float32)],
)


def _hist_body(sums_hbm, minv_hbm, rngv_hbm, out_hbm, buf, minb, rngb, hist):
    wid = lax.axis_index("s") * NC + lax.axis_index("c")
    pltpu.sync_copy(sums_hbm.at[wid], buf)
    pltpu.sync_copy(minv_hbm, minb)
    pltpu.sync_copy(rngv_hbm, rngb)
    zeros16 = jnp.zeros((L,), jnp.float32)
    ones16 = jnp.ones((L,), jnp.float32)
    lanes = lax.iota(jnp.int32, L)

    def zb(b, _):
        hist[pl.ds(b * L, L)] = zeros16
        return 0

    lax.fori_loop(0, NPART * NBINS2, zb, 0)

    def body(c, _):
        col = c * L
        for p in range(NPART):
            x = buf[2 * p, pl.ds(col, L)]
            y = buf[2 * p + 1, pl.ds(col, L)]
            xn = (x - minb[2 * p]) / rngb[2 * p]
            yn = (y - minb[2 * p + 1]) / rngb[2 * p + 1]
            xb = jnp.clip((xn * float(NB)).astype(jnp.int32), 0, NB - 1)
            yb = jnp.clip((yn * float(NB)).astype(jnp.int32), 0, NB - 1)
            bi = (p * NBINS2 + xb * NB + yb) * L + lanes
            plsc.addupdate_scatter(hist, [bi], ones16)
        return 0

    lax.fori_loop(0, RPW // L, body, 0)
    pltpu.sync_copy(hist, out_hbm.at[wid])


def _make_hist_call():
    return pl.kernel(
        _hist_body,
        out_type=jax.ShapeDtypeStruct((NW, NPART * NBINS2 * L), jnp.float32),
        mesh=plsc.VectorSubcoreMesh(core_axis_name="c", subcore_axis_name="s",
                                    num_cores=NC, num_subcores=NS),
        scratch_types=[pltpu.VMEM((NSER, RPW), jnp.float32),
                       pltpu.VMEM((NSER, L), jnp.float32),
                       pltpu.VMEM((NSER, L), jnp.float32),
                       pltpu.VMEM((NPART * NBINS2 * L,), jnp.float32)],
        compiler_params=pltpu.CompilerParams(needs_layout_passes=False),
    )


def _mi_body(h_ref, mis_ref, integ_ref):
    h = jnp.sum(h_ref[...], axis=(0, 3))
    total = jnp.sum(h, axis=1, keepdims=True)
    jn = h / (total + 1e-10)
    ki = lax.broadcasted_iota(jnp.int32, (NBINS2, NBINS2), 0)
    li = lax.broadcasted_iota(jnp.int32, (NBINS2, NBINS2), 1)
    m1 = ((ki // NB) == (li // NB)).astype(jnp.float32)
    m2 = ((ki % NB) == (li % NB)).astype(jnp.float32)
    px = lax.dot_general(jn, m1, (((1,), (0,)), ((), ())),
                         preferred_element_type=jnp.float32,
                         precision=lax.Precision.HIGHEST)
    py = lax.dot_general(jn, m2, (((1,), (0,)), ((), ())),
                         preferred_element_type=jnp.float32,
                         precision=lax.Precision.HIGHEST)
    mi = jnp.sum(jn * jnp.log((jn + 1e-10) / (px * py + 1e-10)), axis=1)
    mi = jnp.maximum(mi, 0.0)
    mis_ref[...] = mi[None, :]
    integ_ref[...] = jnp.min(mi).reshape(1, 1)


_mi = pl.pallas_call(
    _mi_body,
    out_shape=[jax.ShapeDtypeStruct((1, NPART), jnp.float32),
               jax.ShapeDtypeStruct((1, 1), jnp.float32)],
)


def kernel(states, partitions):
    mask_f = partitions.astype(jnp.float32)
    masks = jnp.stack([mask_f, 1.0 - mask_f], axis=1).reshape(NSER, D)
    sums_t, mins, maxs = _stage1(states, masks)
    rng = maxs - mins + 1e-6
    minv = jnp.broadcast_to(mins.reshape(NSER, 1), (NSER, L))
    rngv = jnp.broadcast_to(rng.reshape(NSER, 1), (NSER, L))
    hist = _make_hist_call()(sums_t, minv, rngv)
    mis, integ = _mi(hist.reshape(NW, NPART, NBINS2, L))
    return (integ[0, 0], mis[0])


# speedup vs baseline: 4.5780x; 1.6934x over previous
import jax
import jax.numpy as jnp
from jax import lax
from jax.experimental import pallas as pl
from jax.experimental.pallas import tpu as pltpu
from jax.experimental.pallas import tpu_sc as plsc

T = 65536
D = 512
NB = 10
NC, NS, L = 2, 16, 16
NW = NC * NS
RPW = T // NW
NSER = 8
NPART = 4
NBINS2 = NB * NB


BLK = 4096
BPG = BLK // RPW


def _stage1_body(st_ref, m_ref, sums_ref, minv_ref, rngv_ref, acc_ref):
    i = pl.program_id(0)
    m = m_ref[...]
    na = jnp.sum(m, axis=1, keepdims=True)
    mb = m.astype(jnp.bfloat16)
    x = st_ref[...]
    hi = lax.bitcast_convert_type(
        lax.bitcast_convert_type(x, jnp.uint32) & jnp.uint32(0xFFFF0000),
        jnp.float32)
    lo = (x - hi).astype(jnp.bfloat16)
    dn = (((1,), (1,)), ((), ()))
    s = (lax.dot_general(mb, hi.astype(jnp.bfloat16), dn,
                         preferred_element_type=jnp.float32)
         + lax.dot_general(mb, lo, dn,
                           preferred_element_type=jnp.float32))
    s = s / na
    for b in range(BPG):
        sums_ref[b] = s[:, b * RPW:(b + 1) * RPW]
    bmin = jnp.min(s, axis=1, keepdims=True)
    bmax = jnp.max(s, axis=1, keepdims=True)

    @pl.when(i == 0)
    def _():
        acc_ref[:, 0:1] = bmin
        acc_ref[:, 1:2] = bmax

    @pl.when(i != 0)
    def _():
        acc_ref[:, 0:1] = jnp.minimum(acc_ref[:, 0:1], bmin)
        acc_ref[:, 1:2] = jnp.maximum(acc_ref[:, 1:2], bmax)

    @pl.when(i == T // BLK - 1)
    def _():
        mn = acc_ref[:, 0:1]
        rg = acc_ref[:, 1:2] - mn + 1e-6
        minv_ref[...] = jnp.broadcast_to(mn, (NSER, L))
        rngv_ref[...] = jnp.broadcast_to(rg, (NSER, L))


_stage1 = pl.pallas_call(
    _stage1_body,
    grid=(T // BLK,),
    in_specs=[pl.BlockSpec((BLK, D), lambda i: (i, 0)),
              pl.BlockSpec((NSER, D), lambda i: (0, 0))],
    out_specs=[pl.BlockSpec((BPG, NSER, RPW), lambda i: (i, 0, 0)),
               pl.BlockSpec((NSER, L), lambda i: (0, 0)),
               pl.BlockSpec((NSER, L), lambda i: (0, 0))],
    out_shape=[jax.ShapeDtypeStruct((NW, NSER, RPW), jnp.float32),
               jax.ShapeDtypeStruct((NSER, L), jnp.float32),
               jax.ShapeDtypeStruct((NSER, L), jnp.float32)],
    scratch_shapes=[pltpu.VMEM((NSER, 2), jnp.float32)],
)


def _hist_body(sums_hbm, minv_hbm, rngv_hbm, out_hbm, buf, minb, rngb, hist):
    wid = lax.axis_index("s") * NC + lax.axis_index("c")
    pltpu.sync_copy(sums_hbm.at[wid], buf)
    pltpu.sync_copy(minv_hbm, minb)
    pltpu.sync_copy(rngv_hbm, rngb)
    zeros16 = jnp.zeros((L,), jnp.float32)
    ones16 = jnp.ones((L,), jnp.float32)
    lanes = lax.iota(jnp.int32, L)

    def zb(b, _):
        hist[pl.ds(b * L, L)] = zeros16
        return 0

    lax.fori_loop(0, NPART * NBINS2, zb, 0)

    def body(c, _):
        col = c * L
        for p in range(NPART):
            x = buf[2 * p, pl.ds(col, L)]
            y = buf[2 * p + 1, pl.ds(col, L)]
            xn = (x - minb[2 * p]) / rngb[2 * p]
            yn = (y - minb[2 * p + 1]) / rngb[2 * p + 1]
            xb = jnp.clip((xn * float(NB)).astype(jnp.int32), 0, NB - 1)
            yb = jnp.clip((yn * float(NB)).astype(jnp.int32), 0, NB - 1)
            bi = (p * NBINS2 + xb * NB + yb) * L + lanes
            plsc.addupdate_scatter(hist, [bi], ones16)
        return 0

    lax.fori_loop(0, RPW // L, body, 0)
    pltpu.sync_copy(hist, out_hbm.at[wid])


def _make_hist_call():
    return pl.kernel(
        _hist_body,
        out_type=jax.ShapeDtypeStruct((NW, NPART * NBINS2 * L), jnp.float32),
        mesh=plsc.VectorSubcoreMesh(core_axis_name="c", subcore_axis_name="s",
                                    num_cores=NC, num_subcores=NS),
        scratch_types=[pltpu.VMEM((NSER, RPW), jnp.float32),
                       pltpu.VMEM((NSER, L), jnp.float32),
                       pltpu.VMEM((NSER, L), jnp.float32),
                       pltpu.VMEM((NPART * NBINS2 * L,), jnp.float32)],
        compiler_params=pltpu.CompilerParams(needs_layout_passes=False),
    )


def _mi_body(h_ref, mis_ref, integ_ref):
    h = jnp.sum(h_ref[...], axis=(0, 3))
    total = jnp.sum(h, axis=1, keepdims=True)
    jn = h / (total + 1e-10)
    ki = lax.broadcasted_iota(jnp.int32, (NBINS2, NBINS2), 0)
    li = lax.broadcasted_iota(jnp.int32, (NBINS2, NBINS2), 1)
    m1 = ((ki // NB) == (li // NB)).astype(jnp.float32)
    m2 = ((ki % NB) == (li % NB)).astype(jnp.float32)
    px = lax.dot_general(jn, m1, (((1,), (0,)), ((), ())),
                         preferred_element_type=jnp.float32,
                         precision=lax.Precision.HIGHEST)
    py = lax.dot_general(jn, m2, (((1,), (0,)), ((), ())),
                         preferred_element_type=jnp.float32,
                         precision=lax.Precision.HIGHEST)
    mi = jnp.sum(jn * jnp.log((jn + 1e-10) / (px * py + 1e-10)), axis=1)
    mi = jnp.maximum(mi, 0.0)
    mis_ref[...] = mi[None, :]
    integ_ref[...] = jnp.min(mi).reshape(1, 1)


_mi = pl.pallas_call(
    _mi_body,
    out_shape=[jax.ShapeDtypeStruct((1, NPART), jnp.float32),
               jax.ShapeDtypeStruct((1, 1), jnp.float32)],
)


def kernel(states, partitions):
    mask_f = partitions.astype(jnp.float32)
    masks = jnp.stack([mask_f, 1.0 - mask_f], axis=1).reshape(NSER, D)
    sums_t, minv, rngv = _stage1(states, masks)
    hist = _make_hist_call()(sums_t, minv, rngv)
    mis, integ = _mi(hist.reshape(NW, NPART, NBINS2, L))
    return (integ[0, 0], mis[0])


# speedup vs baseline: 4.9322x; 1.0774x over previous
import jax
import jax.numpy as jnp
from jax import lax
from jax.experimental import pallas as pl
from jax.experimental.pallas import tpu as pltpu
from jax.experimental.pallas import tpu_sc as plsc

T = 65536
D = 512
NB = 10
NC, NS, L = 2, 16, 16
NW = NC * NS
RPW = T // NW
NSER = 8
NPART = 4
NBINS2 = NB * NB


BLK = 8192
BPG = BLK // RPW


def _stage1_body(st_ref, m_ref, sums_ref, minv_ref, rngv_ref, acc_ref):
    i = pl.program_id(0)
    m = m_ref[...]
    na = jnp.sum(m, axis=1, keepdims=True)
    mb = m.astype(jnp.bfloat16)
    x = st_ref[...]
    hi = lax.bitcast_convert_type(
        lax.bitcast_convert_type(x, jnp.uint32) & jnp.uint32(0xFFFF0000),
        jnp.float32)
    lo = (x - hi).astype(jnp.bfloat16)
    dn = (((1,), (1,)), ((), ()))
    s = (lax.dot_general(mb, hi.astype(jnp.bfloat16), dn,
                         preferred_element_type=jnp.float32)
         + lax.dot_general(mb, lo, dn,
                           preferred_element_type=jnp.float32))
    s = s / na
    for b in range(BPG):
        sums_ref[b] = s[:, b * RPW:(b + 1) * RPW]
    bmin = jnp.min(s, axis=1, keepdims=True)
    bmax = jnp.max(s, axis=1, keepdims=True)

    @pl.when(i == 0)
    def _():
        acc_ref[:, 0:1] = bmin
        acc_ref[:, 1:2] = bmax

    @pl.when(i != 0)
    def _():
        acc_ref[:, 0:1] = jnp.minimum(acc_ref[:, 0:1], bmin)
        acc_ref[:, 1:2] = jnp.maximum(acc_ref[:, 1:2], bmax)

    @pl.when(i == T // BLK - 1)
    def _():
        mn = acc_ref[:, 0:1]
        rg = acc_ref[:, 1:2] - mn + 1e-6
        minv_ref[...] = jnp.broadcast_to(mn, (NSER, L))
        rngv_ref[...] = jnp.broadcast_to(float(NB) / rg, (NSER, L))


_stage1 = pl.pallas_call(
    _stage1_body,
    grid=(T // BLK,),
    in_specs=[pl.BlockSpec((BLK, D), lambda i: (i, 0)),
              pl.BlockSpec((NSER, D), lambda i: (0, 0))],
    out_specs=[pl.BlockSpec((BPG, NSER, RPW), lambda i: (i, 0, 0)),
               pl.BlockSpec((NSER, L), lambda i: (0, 0)),
               pl.BlockSpec((NSER, L), lambda i: (0, 0))],
    out_shape=[jax.ShapeDtypeStruct((NW, NSER, RPW), jnp.float32),
               jax.ShapeDtypeStruct((NSER, L), jnp.float32),
               jax.ShapeDtypeStruct((NSER, L), jnp.float32)],
    scratch_shapes=[pltpu.VMEM((NSER, 2), jnp.float32)],
)


def _hist_body(sums_hbm, minv_hbm, s10v_hbm, out_hbm, buf, minb, s10b, hist):
    wid = lax.axis_index("s") * NC + lax.axis_index("c")
    pltpu.sync_copy(sums_hbm.at[wid], buf)
    pltpu.sync_copy(minv_hbm, minb)
    pltpu.sync_copy(s10v_hbm, s10b)
    zeros16 = jnp.zeros((L,), jnp.float32)
    ones16 = jnp.ones((L,), jnp.float32)
    lanes = lax.iota(jnp.int32, L)

    def zb(b, _):
        for u in range(4):
            hist[pl.ds((b * 4 + u) * L, L)] = zeros16
        return 0

    lax.fori_loop(0, NPART * NBINS2 // 4, zb, 0)

    mns = [minb[k] for k in range(NSER)]
    s10 = [s10b[k] for k in range(NSER)]

    UNROLL = 4

    def body(c, _):
        for u in range(UNROLL):
            col = (c * UNROLL + u) * L
            for p in range(NPART):
                x = buf[2 * p, pl.ds(col, L)]
                y = buf[2 * p + 1, pl.ds(col, L)]
                xb = jnp.clip(((x - mns[2 * p]) * s10[2 * p])
                              .astype(jnp.int32), 0, NB - 1)
                yb = jnp.clip(((y - mns[2 * p + 1]) * s10[2 * p + 1])
                              .astype(jnp.int32), 0, NB - 1)
                bi = (p * NBINS2 + xb * NB + yb) * L + lanes
                plsc.addupdate_scatter(hist, [bi], ones16)
        return 0

    lax.fori_loop(0, RPW // L // UNROLL, body, 0)
    pltpu.sync_copy(hist, out_hbm.at[wid])


def _make_hist_call():
    return pl.kernel(
        _hist_body,
        out_type=jax.ShapeDtypeStruct((NW, NPART * NBINS2 * L), jnp.float32),
        mesh=plsc.VectorSubcoreMesh(core_axis_name="c", subcore_axis_name="s",
                                    num_cores=NC, num_subcores=NS),
        scratch_types=[pltpu.VMEM((NSER, RPW), jnp.float32),
                       pltpu.VMEM((NSER, L), jnp.float32),
                       pltpu.VMEM((NSER, L), jnp.float32),
                       pltpu.VMEM((NPART * NBINS2 * L,), jnp.float32)],
        compiler_params=pltpu.CompilerParams(needs_layout_passes=False),
    )


def _mi_body(h_ref, mis_ref, integ_ref):
    h = jnp.sum(h_ref[...], axis=(0, 3))
    total = jnp.sum(h, axis=1, keepdims=True)
    jn = h / (total + 1e-10)
    ki = lax.broadcasted_iota(jnp.int32, (NBINS2, NBINS2), 0)
    li = lax.broadcasted_iota(jnp.int32, (NBINS2, NBINS2), 1)
    m1 = ((ki // NB) == (li // NB)).astype(jnp.float32)
    m2 = ((ki % NB) == (li % NB)).astype(jnp.float32)
    px = lax.dot_general(jn, m1, (((1,), (0,)), ((), ())),
                         preferred_element_type=jnp.float32,
                         precision=lax.Precision.HIGHEST)
    py = lax.dot_general(jn, m2, (((1,), (0,)), ((), ())),
                         preferred_element_type=jnp.float32,
                         precision=lax.Precision.HIGHEST)
    mi = jnp.sum(jn * jnp.log((jn + 1e-10) / (px * py + 1e-10)), axis=1)
    mi = jnp.maximum(mi, 0.0)
    mis_ref[...] = mi[None, :]
    integ_ref[...] = jnp.min(mi).reshape(1, 1)


_mi = pl.pallas_call(
    _mi_body,
    out_shape=[jax.ShapeDtypeStruct((1, NPART), jnp.float32),
               jax.ShapeDtypeStruct((1, 1), jnp.float32)],
)


def kernel(states, partitions):
    mask_f = partitions.astype(jnp.float32)
    masks = jnp.stack([mask_f, 1.0 - mask_f], axis=1).reshape(NSER, D)
    sums_t, minv, s10v = _stage1(states, masks)
    hist = _make_hist_call()(sums_t, minv, s10v)
    mis, integ = _mi(hist.reshape(NW, NPART, NBINS2, L))
    return (integ[0, 0], mis[0])


# speedup vs baseline: 5.2761x; 1.0697x over previous
import jax
import jax.numpy as jnp
from jax import lax
from jax.experimental import pallas as pl
from jax.experimental.pallas import tpu as pltpu
from jax.experimental.pallas import tpu_sc as plsc

T = 65536
D = 512
NB = 10
NC, NS, L = 2, 16, 16
NW = NC * NS
RPW = T // NW
NSER = 8
NPART = 4
NBINS2 = NB * NB


BLK = 8192
BPG = BLK // RPW


def _stage1_body(st_ref, m_ref, sums_ref, minv_ref, rngv_ref, acc_ref):
    i = pl.program_id(0)
    m = m_ref[...]
    na = jnp.sum(m, axis=1, keepdims=True)
    mb = m.astype(jnp.bfloat16)
    x = st_ref[...]
    hi = lax.bitcast_convert_type(
        lax.bitcast_convert_type(x, jnp.uint32) & jnp.uint32(0xFFFF0000),
        jnp.float32)
    lo = (x - hi).astype(jnp.bfloat16)
    dn = (((1,), (1,)), ((), ()))
    s = (lax.dot_general(mb, hi.astype(jnp.bfloat16), dn,
                         preferred_element_type=jnp.float32)
         + lax.dot_general(mb, lo, dn,
                           preferred_element_type=jnp.float32))
    s = s / na
    for b in range(BPG):
        sums_ref[b] = s[:, b * RPW:(b + 1) * RPW]
    bmin = jnp.min(s, axis=1, keepdims=True)
    bmax = jnp.max(s, axis=1, keepdims=True)

    @pl.when(i == 0)
    def _():
        acc_ref[:, 0:1] = bmin
        acc_ref[:, 1:2] = bmax

    @pl.when(i != 0)
    def _():
        acc_ref[:, 0:1] = jnp.minimum(acc_ref[:, 0:1], bmin)
        acc_ref[:, 1:2] = jnp.maximum(acc_ref[:, 1:2], bmax)

    @pl.when(i == T // BLK - 1)
    def _():
        mn = acc_ref[:, 0:1]
        rg = acc_ref[:, 1:2] - mn + 1e-6
        minv_ref[...] = jnp.broadcast_to(mn, (NSER, L))
        rngv_ref[...] = jnp.broadcast_to(float(NB) / rg, (NSER, L))


_stage1 = pl.pallas_call(
    _stage1_body,
    grid=(T // BLK,),
    in_specs=[pl.BlockSpec((BLK, D), lambda i: (i, 0)),
              pl.BlockSpec((NSER, D), lambda i: (0, 0))],
    out_specs=[pl.BlockSpec((BPG, NSER, RPW), lambda i: (i, 0, 0)),
               pl.BlockSpec((NSER, L), lambda i: (0, 0)),
               pl.BlockSpec((NSER, L), lambda i: (0, 0))],
    out_shape=[jax.ShapeDtypeStruct((NW, NSER, RPW), jnp.float32),
               jax.ShapeDtypeStruct((NSER, L), jnp.float32),
               jax.ShapeDtypeStruct((NSER, L), jnp.float32)],
    scratch_shapes=[pltpu.VMEM((NSER, 2), jnp.float32)],
)


def _hist_body(sums_hbm, minv_hbm, s10v_hbm, out_hbm, buf, minb, s10b, hist):
    wid = lax.axis_index("s") * NC + lax.axis_index("c")
    pltpu.sync_copy(sums_hbm.at[wid], buf)
    pltpu.sync_copy(minv_hbm, minb)
    pltpu.sync_copy(s10v_hbm, s10b)
    zeros16 = jnp.zeros((L,), jnp.float32)
    ones16 = jnp.ones((L,), jnp.float32)
    lanes = lax.iota(jnp.int32, L)

    @plsc.parallel_loop(0, NPART * NBINS2, step=1, unroll=8)
    def _zero(b):
        hist[pl.ds(b * L, L)] = zeros16

    mns = [minb[k] for k in range(NSER)]
    s10 = [s10b[k] for k in range(NSER)]

    @plsc.parallel_loop(0, RPW // L, step=1, unroll=4)
    def _accum(c):
        col = c * L
        for p in range(NPART):
            x = buf[2 * p, pl.ds(col, L)]
            y = buf[2 * p + 1, pl.ds(col, L)]
            xb = jnp.clip(((x - mns[2 * p]) * s10[2 * p])
                          .astype(jnp.int32), 0, NB - 1)
            yb = jnp.clip(((y - mns[2 * p + 1]) * s10[2 * p + 1])
                          .astype(jnp.int32), 0, NB - 1)
            bi = (p * NBINS2 + xb * NB + yb) * L + lanes
            plsc.addupdate_scatter(hist, [bi], ones16)

    pltpu.sync_copy(hist, out_hbm.at[wid])


def _make_hist_call():
    return pl.kernel(
        _hist_body,
        out_type=jax.ShapeDtypeStruct((NW, NPART * NBINS2 * L), jnp.float32),
        mesh=plsc.VectorSubcoreMesh(core_axis_name="c", subcore_axis_name="s",
                                    num_cores=NC, num_subcores=NS),
        scratch_types=[pltpu.VMEM((NSER, RPW), jnp.float32),
                       pltpu.VMEM((NSER, L), jnp.float32),
                       pltpu.VMEM((NSER, L), jnp.float32),
                       pltpu.VMEM((NPART * NBINS2 * L,), jnp.float32)],
        compiler_params=pltpu.CompilerParams(needs_layout_passes=False),
    )


def _mi_body(h_ref, mis_ref, integ_ref):
    h = jnp.sum(h_ref[...], axis=(0, 3))
    total = jnp.sum(h, axis=1, keepdims=True)
    jn = h / (total + 1e-10)
    ki = lax.broadcasted_iota(jnp.int32, (NBINS2, NBINS2), 0)
    li = lax.broadcasted_iota(jnp.int32, (NBINS2, NBINS2), 1)
    m1 = ((ki // NB) == (li // NB)).astype(jnp.float32)
    m2 = ((ki % NB) == (li % NB)).astype(jnp.float32)
    px = lax.dot_general(jn, m1, (((1,), (0,)), ((), ())),
                         preferred_element_type=jnp.float32,
                         precision=lax.Precision.HIGHEST)
    py = lax.dot_general(jn, m2, (((1,), (0,)), ((), ())),
                         preferred_element_type=jnp.float32,
                         precision=lax.Precision.HIGHEST)
    mi = jnp.sum(jn * jnp.log((jn + 1e-10) / (px * py + 1e-10)), axis=1)
    mi = jnp.maximum(mi, 0.0)
    mis_ref[...] = mi[None, :]
    integ_ref[...] = jnp.min(mi).reshape(1, 1)


_mi = pl.pallas_call(
    _mi_body,
    out_shape=[jax.ShapeDtypeStruct((1, NPART), jnp.float32),
               jax.ShapeDtypeStruct((1, 1), jnp.float32)],
)


def kernel(states, partitions):
    mask_f = partitions.astype(jnp.float32)
    masks = jnp.stack([mask_f, 1.0 - mask_f], axis=1).reshape(NSER, D)
    sums_t, minv, s10v = _stage1(states, masks)
    hist = _make_hist_call()(sums_t, minv, s10v)
    mis, integ = _mi(hist.reshape(NW, NPART, NBINS2, L))
    return (integ[0, 0], mis[0])


# speedup vs baseline: 5.2845x; 1.0016x over previous
import jax
import jax.numpy as jnp
from jax import lax
from jax.experimental import pallas as pl
from jax.experimental.pallas import tpu as pltpu
from jax.experimental.pallas import tpu_sc as plsc

T = 65536
D = 512
NB = 10
NC, NS, L = 2, 16, 16
NW = NC * NS
RPW = T // NW
NSER = 8
NPART = 4
NBINS2 = NB * NB


BLK = 8192
BPG = BLK // RPW


def _stage1_body(st_ref, m_ref, sums_ref, minv_ref, rngv_ref, acc_ref):
    i = pl.program_id(0)
    m = m_ref[...]
    na = jnp.sum(m, axis=1, keepdims=True)
    mb = m.astype(jnp.bfloat16)
    x = st_ref[...]
    hi = lax.bitcast_convert_type(
        lax.bitcast_convert_type(x, jnp.uint32) & jnp.uint32(0xFFFF0000),
        jnp.float32)
    lo = (x - hi).astype(jnp.bfloat16)
    dn = (((1,), (1,)), ((), ()))
    s = (lax.dot_general(mb, hi.astype(jnp.bfloat16), dn,
                         preferred_element_type=jnp.float32)
         + lax.dot_general(mb, lo, dn,
                           preferred_element_type=jnp.float32))
    s = s / na
    for b in range(BPG):
        sums_ref[b] = s[:, b * RPW:(b + 1) * RPW]
    bmin = jnp.min(s, axis=1, keepdims=True)
    bmax = jnp.max(s, axis=1, keepdims=True)

    @pl.when(i == 0)
    def _():
        acc_ref[:, 0:1] = bmin
        acc_ref[:, 1:2] = bmax

    @pl.when(i != 0)
    def _():
        acc_ref[:, 0:1] = jnp.minimum(acc_ref[:, 0:1], bmin)
        acc_ref[:, 1:2] = jnp.maximum(acc_ref[:, 1:2], bmax)

    @pl.when(i == T // BLK - 1)
    def _():
        mn = acc_ref[:, 0:1]
        rg = acc_ref[:, 1:2] - mn + 1e-6
        minv_ref[...] = jnp.broadcast_to(mn, (NSER, L))
        rngv_ref[...] = jnp.broadcast_to(float(NB) / rg, (NSER, L))


_stage1 = pl.pallas_call(
    _stage1_body,
    grid=(T // BLK,),
    in_specs=[pl.BlockSpec((BLK, D), lambda i: (i, 0)),
              pl.BlockSpec((NSER, D), lambda i: (0, 0))],
    out_specs=[pl.BlockSpec((BPG, NSER, RPW), lambda i: (i, 0, 0)),
               pl.BlockSpec((NSER, L), lambda i: (0, 0)),
               pl.BlockSpec((NSER, L), lambda i: (0, 0))],
    out_shape=[jax.ShapeDtypeStruct((NW, NSER, RPW), jnp.float32),
               jax.ShapeDtypeStruct((NSER, L), jnp.float32),
               jax.ShapeDtypeStruct((NSER, L), jnp.float32)],
    scratch_shapes=[pltpu.VMEM((NSER, 2), jnp.float32)],
)


def _hist_body(sums_hbm, minv_hbm, s10v_hbm, out_hbm, buf, minb, s10b, hist):
    wid = lax.axis_index("s") * NC + lax.axis_index("c")
    pltpu.sync_copy(sums_hbm.at[wid], buf)
    pltpu.sync_copy(minv_hbm, minb)
    pltpu.sync_copy(s10v_hbm, s10b)
    zeros16 = jnp.zeros((L,), jnp.float32)
    ones16 = jnp.ones((L,), jnp.float32)
    lanes = lax.iota(jnp.int32, L)

    @plsc.parallel_loop(0, NPART * NBINS2, step=1, unroll=8)
    def _zero(b):
        hist[pl.ds(b * L, L)] = zeros16

    mns = [minb[k] for k in range(NSER)]
    s10 = [s10b[k] for k in range(NSER)]

    @plsc.parallel_loop(0, RPW // L, step=1, unroll=8)
    def _accum(c):
        col = c * L
        for p in range(NPART):
            x = buf[2 * p, pl.ds(col, L)]
            y = buf[2 * p + 1, pl.ds(col, L)]
            xb = jnp.minimum(((x - mns[2 * p]) * s10[2 * p])
                             .astype(jnp.int32), NB - 1)
            yb = jnp.minimum(((y - mns[2 * p + 1]) * s10[2 * p + 1])
                             .astype(jnp.int32), NB - 1)
            bi = (p * NBINS2 + xb * NB + yb) * L + lanes
            plsc.addupdate_scatter(hist, [bi], ones16)

    pltpu.sync_copy(hist, out_hbm.at[wid])


def _make_hist_call():
    return pl.kernel(
        _hist_body,
        out_type=jax.ShapeDtypeStruct((NW, NPART * NBINS2 * L), jnp.float32),
        mesh=plsc.VectorSubcoreMesh(core_axis_name="c", subcore_axis_name="s",
                                    num_cores=NC, num_subcores=NS),
        scratch_types=[pltpu.VMEM((NSER, RPW), jnp.float32),
                       pltpu.VMEM((NSER, L), jnp.float32),
                       pltpu.VMEM((NSER, L), jnp.float32),
                       pltpu.VMEM((NPART * NBINS2 * L,), jnp.float32)],
        compiler_params=pltpu.CompilerParams(needs_layout_passes=False),
    )


def _mi_body(h_ref, mis_ref, integ_ref):
    h = jnp.sum(h_ref[...], axis=(0, 3))
    total = jnp.sum(h, axis=1, keepdims=True)
    jn = h / (total + 1e-10)
    ki = lax.broadcasted_iota(jnp.int32, (NBINS2, NBINS2), 0)
    li = lax.broadcasted_iota(jnp.int32, (NBINS2, NBINS2), 1)
    m1 = ((ki // NB) == (li // NB)).astype(jnp.float32)
    m2 = ((ki % NB) == (li % NB)).astype(jnp.float32)
    px = lax.dot_general(jn, m1, (((1,), (0,)), ((), ())),
                         preferred_element_type=jnp.float32,
                         precision=lax.Precision.HIGHEST)
    py = lax.dot_general(jn, m2, (((1,), (0,)), ((), ())),
                         preferred_element_type=jnp.float32,
                         precision=lax.Precision.HIGHEST)
    mi = jnp.sum(jn * jnp.log((jn + 1e-10) / (px * py + 1e-10)), axis=1)
    mi = jnp.maximum(mi, 0.0)
    mis_ref[...] = mi[None, :]
    integ_ref[...] = jnp.min(mi).reshape(1, 1)


_mi = pl.pallas_call(
    _mi_body,
    out_shape=[jax.ShapeDtypeStruct((1, NPART), jnp.float32),
               jax.ShapeDtypeStruct((1, 1), jnp.float32)],
)


def kernel(states, partitions):
    mask_f = partitions.astype(jnp.float32)
    masks = jnp.stack([mask_f, 1.0 - mask_f], axis=1).reshape(NSER, D)
    sums_t, minv, s10v = _stage1(states, masks)
    hist = _make_hist_call()(sums_t, minv, s10v)
    mis, integ = _mi(hist.reshape(NW, NPART, NBINS2, L))
    return (integ[0, 0], mis[0])


# speedup vs baseline: 5.6899x; 1.0767x over previous
import jax
import jax.numpy as jnp
from jax import lax
from jax.experimental import pallas as pl
from jax.experimental.pallas import tpu as pltpu
from jax.experimental.pallas import tpu_sc as plsc

T = 65536
D = 512
NB = 10
NC, NS, L = 2, 16, 16
NW = NC * NS
RPW = T // NW
NSER = 8
NPART = 4
NBINS2 = NB * NB


BLK = 8192
BPG = BLK // RPW


def _stage1_body(st_ref, m_ref, sums_ref, minv_ref, rngv_ref, acc_ref):
    i = pl.program_id(0)
    m = m_ref[...]
    na = jnp.sum(m, axis=1, keepdims=True)
    mb = m.astype(jnp.bfloat16)
    x = st_ref[...]
    hi = lax.bitcast_convert_type(
        lax.bitcast_convert_type(x, jnp.uint32) & jnp.uint32(0xFFFF0000),
        jnp.float32)
    lo = (x - hi).astype(jnp.bfloat16)
    dn = (((1,), (1,)), ((), ()))
    s = (lax.dot_general(mb, hi.astype(jnp.bfloat16), dn,
                         preferred_element_type=jnp.float32)
         + lax.dot_general(mb, lo, dn,
                           preferred_element_type=jnp.float32))
    s = s / na
    for b in range(BPG):
        sums_ref[b] = s[:, b * RPW:(b + 1) * RPW]
    bmin = jnp.min(s, axis=1, keepdims=True)
    bmax = jnp.max(s, axis=1, keepdims=True)

    @pl.when(i == 0)
    def _():
        acc_ref[:, 0:1] = bmin
        acc_ref[:, 1:2] = bmax

    @pl.when(i != 0)
    def _():
        acc_ref[:, 0:1] = jnp.minimum(acc_ref[:, 0:1], bmin)
        acc_ref[:, 1:2] = jnp.maximum(acc_ref[:, 1:2], bmax)

    @pl.when(i == T // BLK - 1)
    def _():
        mn = acc_ref[:, 0:1]
        rg = acc_ref[:, 1:2] - mn + 1e-6
        minv_ref[...] = jnp.broadcast_to(mn, (NSER, L))
        rngv_ref[...] = jnp.broadcast_to(float(NB) / rg, (NSER, L))


_stage1 = pl.pallas_call(
    _stage1_body,
    grid=(T // BLK,),
    in_specs=[pl.BlockSpec((BLK, D), lambda i: (i, 0)),
              pl.BlockSpec((NSER, D), lambda i: (0, 0))],
    out_specs=[pl.BlockSpec((BPG, NSER, RPW), lambda i: (i, 0, 0)),
               pl.BlockSpec((NSER, L), lambda i: (0, 0)),
               pl.BlockSpec((NSER, L), lambda i: (0, 0))],
    out_shape=[jax.ShapeDtypeStruct((NW, NSER, RPW), jnp.float32),
               jax.ShapeDtypeStruct((NSER, L), jnp.float32),
               jax.ShapeDtypeStruct((NSER, L), jnp.float32)],
    scratch_shapes=[pltpu.VMEM((NSER, 2), jnp.float32)],
)


def _hist_body(sums_hbm, minv_hbm, s10v_hbm, out_hbm, buf, minb, s10b, hist):
    wid = lax.axis_index("s") * NC + lax.axis_index("c")
    pltpu.sync_copy(sums_hbm.at[wid], buf)
    pltpu.sync_copy(minv_hbm, minb)
    pltpu.sync_copy(s10v_hbm, s10b)
    zeros16 = jnp.zeros((L,), jnp.float32)
    ones16 = jnp.ones((L,), jnp.float32)
    lanes = lax.iota(jnp.int32, L)

    @plsc.parallel_loop(0, NPART * NBINS2, step=1, unroll=8)
    def _zero(b):
        hist[pl.ds(b * L, L)] = zeros16

    mns = [minb[k] for k in range(NSER)]
    s10 = [s10b[k] for k in range(NSER)]
    lanebase = lanes * (NPART * NBINS2)

    @plsc.parallel_loop(0, RPW // L, step=1, unroll=8)
    def _accum(c):
        col = c * L
        for p in range(NPART):
            x = buf[2 * p, pl.ds(col, L)]
            y = buf[2 * p + 1, pl.ds(col, L)]
            xb = jnp.minimum(((x - mns[2 * p]) * s10[2 * p])
                             .astype(jnp.int32), NB - 1)
            yb = jnp.minimum(((y - mns[2 * p + 1]) * s10[2 * p + 1])
                             .astype(jnp.int32), NB - 1)
            bi = lanebase + (p * NBINS2 + xb * NB + yb)
            plsc.addupdate_scatter(hist, [bi], ones16)

    pltpu.sync_copy(hist, out_hbm.at[wid])


def _make_hist_call():
    return pl.kernel(
        _hist_body,
        out_type=jax.ShapeDtypeStruct((NW, NPART * NBINS2 * L), jnp.float32),
        mesh=plsc.VectorSubcoreMesh(core_axis_name="c", subcore_axis_name="s",
                                    num_cores=NC, num_subcores=NS),
        scratch_types=[pltpu.VMEM((NSER, RPW), jnp.float32),
                       pltpu.VMEM((NSER, L), jnp.float32),
                       pltpu.VMEM((NSER, L), jnp.float32),
                       pltpu.VMEM((NPART * NBINS2 * L,), jnp.float32)],
        compiler_params=pltpu.CompilerParams(needs_layout_passes=False),
    )


NPB = NPART * NBINS2


def _dot(a, b):
    return lax.dot_general(a, b, (((1,), (0,)), ((), ())),
                           preferred_element_type=jnp.float32,
                           precision=lax.Precision.HIGHEST)


def _mi_body(h_ref, mis_ref, integ_ref):
    h = jnp.sum(h_ref[...], axis=0, keepdims=True)
    h400 = h[:, 0:NPB]
    for l in range(1, L):
        h400 = h400 + h[:, l * NPB:(l + 1) * NPB]
    ki = lax.broadcasted_iota(jnp.int32, (NPB, NPB), 0)
    li = lax.broadcasted_iota(jnp.int32, (NPB, NPB), 1)
    same_p = (ki // NBINS2) == (li // NBINS2)
    mpart = same_p.astype(jnp.float32)
    m1 = (same_p & ((ki % NBINS2) // NB == (li % NBINS2) // NB)
          ).astype(jnp.float32)
    m2 = (same_p & (ki % NB == li % NB)).astype(jnp.float32)
    tot = _dot(h400, mpart)
    jn = h400 / (tot + 1e-10)
    px = _dot(jn, m1)
    py = _dot(jn, m2)
    terms = jn * jnp.log((jn + 1e-10) / (px * py + 1e-10))
    pp = (lax.broadcasted_iota(jnp.int32, (NPB, NPART), 0) // NBINS2 ==
          lax.broadcasted_iota(jnp.int32, (NPB, NPART), 1)).astype(jnp.float32)
    mi = jnp.maximum(_dot(terms, pp), 0.0)
    mis_ref[...] = mi
    integ_ref[...] = jnp.min(mi).reshape(1, 1)


_mi = pl.pallas_call(
    _mi_body,
    out_shape=[jax.ShapeDtypeStruct((1, NPART), jnp.float32),
               jax.ShapeDtypeStruct((1, 1), jnp.float32)],
)


def kernel(states, partitions):
    mask_f = partitions.astype(jnp.float32)
    masks = jnp.stack([mask_f, 1.0 - mask_f], axis=1).reshape(NSER, D)
    sums_t, minv, s10v = _stage1(states, masks)
    hist = _make_hist_call()(sums_t, minv, s10v)
    mis, integ = _mi(hist)
    return (integ[0, 0], mis[0])


# speedup vs baseline: 5.9102x; 1.0387x over previous
import jax
import jax.numpy as jnp
from jax import lax
from jax.experimental import pallas as pl
from jax.experimental.pallas import tpu as pltpu
from jax.experimental.pallas import tpu_sc as plsc

T = 65536
D = 512
NB = 10
NC, NS, L = 2, 16, 16
NW = NC * NS
RPW = T // NW
NSER = 8
NPART = 4
NBINS2 = NB * NB


BLK = 8192
BPG = BLK // RPW


def _stage1_body(st_ref, m_ref, sums_ref, minv_ref, acc_ref):
    i = pl.program_id(0)
    m = m_ref[...]
    na = jnp.sum(m, axis=1, keepdims=True)
    mb = m.astype(jnp.bfloat16)
    x = st_ref[...]
    hi = lax.bitcast_convert_type(
        lax.bitcast_convert_type(x, jnp.uint32) & jnp.uint32(0xFFFF0000),
        jnp.float32)
    lo = (x - hi).astype(jnp.bfloat16)
    dn = (((1,), (1,)), ((), ()))
    s = (lax.dot_general(mb, hi.astype(jnp.bfloat16), dn,
                         preferred_element_type=jnp.float32)
         + lax.dot_general(mb, lo, dn,
                           preferred_element_type=jnp.float32))
    s = s / na
    for b in range(BPG):
        sums_ref[b] = s[:, b * RPW:(b + 1) * RPW]
    bmin = jnp.min(s, axis=1, keepdims=True)
    bmax = jnp.max(s, axis=1, keepdims=True)

    @pl.when(i == 0)
    def _():
        acc_ref[:, 0:1] = bmin
        acc_ref[:, 1:2] = bmax

    @pl.when(i != 0)
    def _():
        acc_ref[:, 0:1] = jnp.minimum(acc_ref[:, 0:1], bmin)
        acc_ref[:, 1:2] = jnp.maximum(acc_ref[:, 1:2], bmax)

    @pl.when(i == T // BLK - 1)
    def _():
        mn = acc_ref[:, 0:1]
        rg = acc_ref[:, 1:2] - mn + 1e-6
        minv_ref[0:NSER] = jnp.broadcast_to(mn, (NSER, L))
        minv_ref[NSER:2 * NSER] = jnp.broadcast_to(float(NB) / rg, (NSER, L))


_stage1 = pl.pallas_call(
    _stage1_body,
    grid=(T // BLK,),
    in_specs=[pl.BlockSpec((BLK, D), lambda i: (i, 0)),
              pl.BlockSpec((NSER, D), lambda i: (0, 0))],
    out_specs=[pl.BlockSpec((BPG, NSER, RPW), lambda i: (i, 0, 0)),
               pl.BlockSpec((2 * NSER, L), lambda i: (0, 0))],
    out_shape=[jax.ShapeDtypeStruct((NW, NSER, RPW), jnp.float32),
               jax.ShapeDtypeStruct((2 * NSER, L), jnp.float32)],
    scratch_shapes=[pltpu.VMEM((NSER, 2), jnp.float32)],
)


def _hist_body(sums_hbm, norm_hbm, out_hbm, buf, normb, hist, sem1, sem2):
    wid = lax.axis_index("s") * NC + lax.axis_index("c")
    c1 = pltpu.async_copy(sums_hbm.at[wid], buf, sem1)
    c2 = pltpu.async_copy(norm_hbm, normb, sem2)
    zeros16 = jnp.zeros((L,), jnp.float32)
    ones16 = jnp.ones((L,), jnp.float32)
    lanes = lax.iota(jnp.int32, L)

    @plsc.parallel_loop(0, NPART * NBINS2, step=1, unroll=8)
    def _zero(b):
        hist[pl.ds(b * L, L)] = zeros16

    c2.wait()
    mns = [normb[k] for k in range(NSER)]
    s10 = [normb[NSER + k] for k in range(NSER)]
    c1.wait()
    lanebase = lanes * (NPART * NBINS2)

    @plsc.parallel_loop(0, RPW // L, step=1, unroll=8)
    def _accum(c):
        col = c * L
        for p in range(NPART):
            x = buf[2 * p, pl.ds(col, L)]
            y = buf[2 * p + 1, pl.ds(col, L)]
            xb = jnp.minimum(((x - mns[2 * p]) * s10[2 * p])
                             .astype(jnp.int32), NB - 1)
            yb = jnp.minimum(((y - mns[2 * p + 1]) * s10[2 * p + 1])
                             .astype(jnp.int32), NB - 1)
            bi = lanebase + (p * NBINS2 + xb * NB + yb)
            plsc.addupdate_scatter(hist, [bi], ones16)

    pltpu.sync_copy(hist, out_hbm.at[wid])


def _make_hist_call():
    return pl.kernel(
        _hist_body,
        out_type=jax.ShapeDtypeStruct((NW, NPART * NBINS2 * L), jnp.float32),
        mesh=plsc.VectorSubcoreMesh(core_axis_name="c", subcore_axis_name="s",
                                    num_cores=NC, num_subcores=NS),
        scratch_types=[pltpu.VMEM((NSER, RPW), jnp.float32),
                       pltpu.VMEM((2 * NSER, L), jnp.float32),
                       pltpu.VMEM((NPART * NBINS2 * L,), jnp.float32),
                       pltpu.SemaphoreType.DMA,
                       pltpu.SemaphoreType.DMA],
        compiler_params=pltpu.CompilerParams(needs_layout_passes=False),
    )


NPB = NPART * NBINS2


def _dot(a, b):
    return lax.dot_general(a, b, (((1,), (0,)), ((), ())),
                           preferred_element_type=jnp.float32,
                           precision=lax.Precision.HIGHEST)


def _mi_body(h_ref, mis_ref, integ_ref):
    h = jnp.sum(h_ref[...], axis=0, keepdims=True)
    h400 = h[:, 0:NPB]
    for l in range(1, L):
        h400 = h400 + h[:, l * NPB:(l + 1) * NPB]
    ki = lax.broadcasted_iota(jnp.int32, (NPB, NPB), 0)
    li = lax.broadcasted_iota(jnp.int32, (NPB, NPB), 1)
    same_p = (ki // NBINS2) == (li // NBINS2)
    mpart = same_p.astype(jnp.float32)
    m1 = (same_p & ((ki % NBINS2) // NB == (li % NBINS2) // NB)
          ).astype(jnp.float32)
    m2 = (same_p & (ki % NB == li % NB)).astype(jnp.float32)
    tot = _dot(h400, mpart)
    jn = h400 / (tot + 1e-10)
    px = _dot(jn, m1)
    py = _dot(jn, m2)
    terms = jn * jnp.log((jn + 1e-10) / (px * py + 1e-10))
    pp = (lax.broadcasted_iota(jnp.int32, (NPB, NPART), 0) // NBINS2 ==
          lax.broadcasted_iota(jnp.int32, (NPB, NPART), 1)).astype(jnp.float32)
    mi = jnp.maximum(_dot(terms, pp), 0.0)
    mis_ref[...] = mi
    integ_ref[...] = jnp.min(mi).reshape(1, 1)


_mi = pl.pallas_call(
    _mi_body,
    out_shape=[jax.ShapeDtypeStruct((1, NPART), jnp.float32),
               jax.ShapeDtypeStruct((1, 1), jnp.float32)],
)


def kernel(states, partitions):
    mask_f = partitions.astype(jnp.float32)
    masks = jnp.stack([mask_f, 1.0 - mask_f], axis=1).reshape(NSER, D)
    sums_t, norm = _stage1(states, masks)
    hist = _make_hist_call()(sums_t, norm)
    mis, integ = _mi(hist)
    return (integ[0, 0], mis[0])
